# baseline (speedup 1.0000x reference)
import functools

import jax
import jax.numpy as jnp
from jax import lax
from jax.experimental import pallas as pl
from jax.experimental.pallas import tpu as pltpu
from jax.experimental.pallas import tpu_sc as plsc

VOCAB = 1000000
EMBED_DIM = 64
BATCH = 16384
HIST = 50

_info = plsc.get_sparse_core_info()
NC = _info.num_cores
NS = _info.num_subcores
NW = NC * NS
ROWS_PER_W = BATCH // NW

GROUP = 8
NBUF = 2
STEPS = ROWS_PER_W // GROUP
OUTER = STEPS // NBUF


@functools.partial(
    pl.kernel,
    mesh=plsc.VectorSubcoreMesh(core_axis_name="c", subcore_axis_name="s"),
    out_type=jax.ShapeDtypeStruct((BATCH, HIST, EMBED_DIM), jnp.float32),
    scratch_types=[
        pltpu.VMEM((ROWS_PER_W, HIST), jnp.int32),
        pltpu.VMEM((NBUF, GROUP, HIST, EMBED_DIM), jnp.float32),
        pltpu.SemaphoreType.DMA,
        pltpu.SemaphoreType.DMA,
        pltpu.SemaphoreType.DMA,
        pltpu.SemaphoreType.DMA,
    ],
    compiler_params=pltpu.CompilerParams(use_tc_tiling_on_sc=False),
)
def _gather_sc(x_hbm, table_hbm, out_hbm, idx_v, rows_v, g0, g1, o0, o1):
    gsem = [g0, g1]
    osem = [o0, o1]
    wid = lax.axis_index("s") * NC + lax.axis_index("c")
    base = wid * ROWS_PER_W
    pltpu.sync_copy(x_hbm.at[pl.ds(base, ROWS_PER_W)], idx_v)

    def issue_gathers(b, row0):
        for i in range(GROUP):
            pltpu.async_copy(
                table_hbm.at[idx_v.at[row0 + i]],
                rows_v.at[b, i],
                gsem[b],
            )

    def drain_gathers(b, row0):
        for i in range(GROUP):
            pltpu.make_async_copy(
                table_hbm.at[idx_v.at[row0 + i]],
                rows_v.at[b, i],
                gsem[b],
            ).wait()

    def out_copy(b, row0):
        return pltpu.make_async_copy(
            rows_v.at[b], out_hbm.at[pl.ds(base + row0, GROUP)], osem[b]
        )

    def step(s, carry):
        rows = [(s * NBUF + b) * GROUP for b in range(NBUF)]
        for b in range(NBUF):
            @pl.when(s > 0)
            def _():
                out_copy(b, rows[b]).wait()

            issue_gathers(b, rows[b])
        for b in range(NBUF):
            drain_gathers(b, rows[b])
            out_copy(b, rows[b]).start()
        return carry

    lax.fori_loop(0, OUTER, step, 0)
    for b in range(NBUF):
        out_copy(b, (STEPS - NBUF + b) * GROUP).wait()


def kernel(x, table):
    return _gather_sc(x, table)


# speedup vs baseline: 1.2087x; 1.2087x over previous
import functools

import jax
import jax.numpy as jnp
from jax import lax
from jax.experimental import pallas as pl
from jax.experimental.pallas import tpu as pltpu
from jax.experimental.pallas import tpu_sc as plsc

VOCAB = 1000000
EMBED_DIM = 64
BATCH = 16384
HIST = 50
HIST_PAD = 56
ROW_PAD = 128

_info = plsc.get_sparse_core_info()
NC = _info.num_cores
NS = _info.num_subcores
NW = NC * NS
ROWS_PER_W = BATCH // NW

GROUP = 4
NBUF = 2
STEPS = ROWS_PER_W // GROUP
OUTER = STEPS // NBUF


@functools.partial(
    pl.kernel,
    mesh=plsc.VectorSubcoreMesh(core_axis_name="c", subcore_axis_name="s"),
    out_type=jax.ShapeDtypeStruct((BATCH, HIST_PAD, ROW_PAD), jnp.float32),
    scratch_types=[
        pltpu.VMEM((ROWS_PER_W, HIST), jnp.int32),
        pltpu.VMEM((NBUF, GROUP, HIST_PAD, ROW_PAD), jnp.float32),
        pltpu.SemaphoreType.DMA,
        pltpu.SemaphoreType.DMA,
        pltpu.SemaphoreType.DMA,
        pltpu.SemaphoreType.DMA,
    ],
    compiler_params=pltpu.CompilerParams(use_tc_tiling_on_sc=False),
)
def _gather_sc(x_hbm, table_hbm, out_hbm, idx_v, rows_v, g0, g1, o0, o1):
    gsem = [g0, g1]
    osem = [o0, o1]
    wid = lax.axis_index("s") * NC + lax.axis_index("c")
    base = wid * ROWS_PER_W
    pltpu.sync_copy(x_hbm.at[pl.ds(base, ROWS_PER_W)], idx_v)

    def issue_gathers(b, row0):
        for i in range(GROUP):
            pltpu.async_copy(
                table_hbm.at[idx_v.at[row0 + i]],
                rows_v.at[b, i, pl.ds(0, HIST)],
                gsem[b],
            )

    def drain_gathers(b, row0):
        for i in range(GROUP):
            pltpu.make_async_copy(
                table_hbm.at[idx_v.at[row0 + i]],
                rows_v.at[b, i, pl.ds(0, HIST)],
                gsem[b],
            ).wait()

    def out_copy(b, row0):
        return pltpu.make_async_copy(
            rows_v.at[b], out_hbm.at[pl.ds(base + row0, GROUP)], osem[b]
        )

    def step(s, carry):
        rows = [(s * NBUF + b) * GROUP for b in range(NBUF)]
        for b in range(NBUF):
            @pl.when(s > 0)
            def _():
                out_copy(b, rows[b]).wait()

            issue_gathers(b, rows[b])
        for b in range(NBUF):
            drain_gathers(b, rows[b])
            out_copy(b, rows[b]).start()
        return carry

    lax.fori_loop(0, OUTER, step, 0)
    for b in range(NBUF):
        out_copy(b, (STEPS - NBUF + b) * GROUP).wait()


def kernel(x, table):
    table_padded = jnp.pad(table, ((0, 0), (0, ROW_PAD - EMBED_DIM)))
    y = _gather_sc(x, table_padded)
    return y[:, :HIST, :EMBED_DIM]


# speedup vs baseline: 1.3489x; 1.1160x over previous
import functools

import jax
import jax.numpy as jnp
from jax import lax
from jax.experimental import pallas as pl
from jax.experimental.pallas import tpu as pltpu
from jax.experimental.pallas import tpu_sc as plsc

VOCAB = 1000000
EMBED_DIM = 64
BATCH = 16384
HIST = 50
HIST_PAD = 56
ROW_PAD = 128

_info = plsc.get_sparse_core_info()
NC = _info.num_cores
NS = _info.num_subcores
NW = NC * NS
ROWS_PER_W = BATCH // NW

GROUP = 8
NBUF = 2
STEPS = ROWS_PER_W // GROUP
OUTER = STEPS // NBUF


@functools.partial(
    pl.kernel,
    mesh=plsc.VectorSubcoreMesh(core_axis_name="c", subcore_axis_name="s"),
    out_type=jax.ShapeDtypeStruct((BATCH, HIST_PAD, ROW_PAD), jnp.float32),
    scratch_types=[
        pltpu.VMEM((ROWS_PER_W, HIST), jnp.int32),
        pltpu.VMEM((NBUF, GROUP, HIST, EMBED_DIM), jnp.float32),
        pltpu.SemaphoreType.DMA,
        pltpu.SemaphoreType.DMA,
        pltpu.SemaphoreType.DMA,
        pltpu.SemaphoreType.DMA,
    ],
    compiler_params=pltpu.CompilerParams(use_tc_tiling_on_sc=False),
)
def _gather_sc(x_hbm, table_hbm, out_hbm, idx_v, rows_v, g0, g1, o0, o1):
    gsem = [g0, g1]
    osem = [o0, o1]
    wid = lax.axis_index("s") * NC + lax.axis_index("c")
    base = wid * ROWS_PER_W
    pltpu.sync_copy(x_hbm.at[pl.ds(base, ROWS_PER_W)], idx_v)

    def issue_gathers(b, row0):
        for i in range(GROUP):
            pltpu.async_copy(
                table_hbm.at[idx_v.at[row0 + i]],
                rows_v.at[b, i],
                gsem[b],
            )

    def drain_gathers(b, row0):
        for i in range(GROUP):
            pltpu.make_async_copy(
                table_hbm.at[idx_v.at[row0 + i]],
                rows_v.at[b, i],
                gsem[b],
            ).wait()

    def out_copy(b, row0):
        return pltpu.make_async_copy(
            rows_v.at[b],
            out_hbm.at[
                pl.ds(base + row0, GROUP), pl.ds(0, HIST), pl.ds(0, EMBED_DIM)
            ],
            osem[b],
        )

    def step(s, carry):
        rows = [(s * NBUF + b) * GROUP for b in range(NBUF)]
        for b in range(NBUF):
            @pl.when(s > 0)
            def _():
                out_copy(b, rows[b]).wait()

            issue_gathers(b, rows[b])
        for b in range(NBUF):
            drain_gathers(b, rows[b])
            out_copy(b, rows[b]).start()
        return carry

    lax.fori_loop(0, OUTER, step, 0)
    for b in range(NBUF):
        out_copy(b, (STEPS - NBUF + b) * GROUP).wait()


def kernel(x, table):
    y = _gather_sc(x, table)
    return y[:, :HIST, :EMBED_DIM]


# speedup vs baseline: 1.3526x; 1.0027x over previous
import functools

import jax
import jax.numpy as jnp
from jax import lax
from jax.experimental import pallas as pl
from jax.experimental.pallas import tpu as pltpu
from jax.experimental.pallas import tpu_sc as plsc

VOCAB = 1000000
EMBED_DIM = 64
BATCH = 16384
HIST = 50
HIST_PAD = 56
ROW_PAD = 128

_info = plsc.get_sparse_core_info()
NC = _info.num_cores
NS = _info.num_subcores
NW = NC * NS
ROWS_PER_W = BATCH // NW

GROUP = 16
NBUF = 2
STEPS = ROWS_PER_W // GROUP
OUTER = STEPS // NBUF


@functools.partial(
    pl.kernel,
    mesh=plsc.VectorSubcoreMesh(core_axis_name="c", subcore_axis_name="s"),
    out_type=jax.ShapeDtypeStruct((BATCH, HIST_PAD, ROW_PAD), jnp.float32),
    scratch_types=[
        pltpu.VMEM((ROWS_PER_W, HIST), jnp.int32),
        pltpu.VMEM((NBUF, GROUP, HIST, EMBED_DIM), jnp.float32),
        pltpu.SemaphoreType.DMA,
        pltpu.SemaphoreType.DMA,
        pltpu.SemaphoreType.DMA,
        pltpu.SemaphoreType.DMA,
    ],
    compiler_params=pltpu.CompilerParams(use_tc_tiling_on_sc=False),
)
def _gather_sc(x_hbm, table_hbm, out_hbm, idx_v, rows_v, g0, g1, o0, o1):
    gsem = [g0, g1]
    osem = [o0, o1]
    wid = lax.axis_index("s") * NC + lax.axis_index("c")
    base = wid * ROWS_PER_W
    pltpu.sync_copy(x_hbm.at[pl.ds(base, ROWS_PER_W)], idx_v)

    def issue_gathers(b, row0):
        for i in range(GROUP):
            pltpu.async_copy(
                table_hbm.at[idx_v.at[row0 + i]],
                rows_v.at[b, i],
                gsem[b],
            )

    def drain_gathers(b, row0):
        for i in range(GROUP):
            pltpu.make_async_copy(
                table_hbm.at[idx_v.at[row0 + i]],
                rows_v.at[b, i],
                gsem[b],
            ).wait()

    def out_copy(b, row0):
        return pltpu.make_async_copy(
            rows_v.at[b],
            out_hbm.at[
                pl.ds(base + row0, GROUP), pl.ds(0, HIST), pl.ds(0, EMBED_DIM)
            ],
            osem[b],
        )

    def step(s, carry):
        rows = [(s * NBUF + b) * GROUP for b in range(NBUF)]
        for b in range(NBUF):
            @pl.when(s > 0)
            def _():
                out_copy(b, rows[b]).wait()

            issue_gathers(b, rows[b])
        for b in range(NBUF):
            drain_gathers(b, rows[b])
            out_copy(b, rows[b]).start()
        return carry

    lax.fori_loop(0, OUTER, step, 0)
    for b in range(NBUF):
        out_copy(b, (STEPS - NBUF + b) * GROUP).wait()


def kernel(x, table):
    y = _gather_sc(x, table)
    return y[:, :HIST, :EMBED_DIM]
